# baseline (speedup 1.0000x reference)
import functools

import jax
import jax.numpy as jnp
from jax import lax
from jax.experimental import pallas as pl
from jax.experimental.pallas import tpu as pltpu
from jax.experimental.pallas import tpu_sc as plsc

_B = 1024
_K = 8
_N = 8192

_NC = 2
_NS = 16
_NW = _NC * _NS

_NH = 2
_FH = _N // _NH
_RPW = _B // (_NW // _NH)
_RG = 4
_NG = _RPW // _RG
_NSLOT = 8
_L = 16
_CH = _FH // _L


def _group_splats(cond_v, row0, nrows):
    splats = []
    for pair in range(nrows // 2):
        v = cond_v[pl.ds((row0 + 2 * pair) * _K, 2 * _K)]
        for r in range(2):
            splats.append([jnp.full((_L,), v[r * _K + k], jnp.float32)
                           for k in range(_K)])
    return splats


def _sc_body(cond_hbm, wt_hbm, b_hbm, out_hbm, cond_v, wt_v, b_v, row_v,
             *sems):
    wid = lax.axis_index("s") * _NC + lax.axis_index("c")
    half = wid % _NH
    rowbase = (wid // _NH) * _RPW
    foff = half * _FH

    pltpu.sync_copy(wt_hbm.at[pl.ds(half * _K * _FH, _K * _FH)], wt_v)
    pltpu.sync_copy(b_hbm.at[pl.ds(half * _FH, _FH)], b_v)
    pltpu.sync_copy(cond_hbm.at[pl.ds(rowbase * _K, _RPW * _K)], cond_v)

    handles = [None] * _NSLOT
    for g in range(_NG):
        rows = [g * _RG + r for r in range(_RG)]
        slots = [row % _NSLOT for row in rows]
        cs = _group_splats(cond_v, g * _RG, _RG)
        for s in slots:
            if handles[s] is not None:
                handles[s].wait()
                handles[s] = None

        def body(j, carry, cs=cs, slots=slots):
            off = j * (2 * _L)
            for u in range(2):
                o = off + u * _L
                bv = b_v[pl.ds(o, _L)]
                wv = [wt_v[pl.ds(k * _FH + o, _L)] for k in range(_K)]
                for r in range(_RG):
                    c = cs[r]
                    a0 = bv + c[0] * wv[0]
                    a0 = a0 + c[1] * wv[1]
                    a0 = a0 + c[2] * wv[2]
                    a0 = a0 + c[3] * wv[3]
                    a1 = c[4] * wv[4] + c[5] * wv[5]
                    a1 = a1 + c[6] * wv[6]
                    a1 = a1 + c[7] * wv[7]
                    row_v[pl.ds(slots[r] * _FH + o, _L)] = a0 + a1
            return carry

        lax.fori_loop(0, _CH // 2, body, 0, unroll=False)

        for r in range(_RG):
            dst = (rowbase + rows[r]) * _N + foff
            handles[slots[r]] = pltpu.async_copy(
                row_v.at[pl.ds(slots[r] * _FH, _FH)],
                out_hbm.at[pl.ds(dst, _FH)],
                sems[slots[r]])

    for s in range(_NSLOT):
        if handles[s] is not None:
            handles[s].wait()


@functools.partial(jax.jit, static_argnames=())
def kernel(condition, W, b):
    wt = W.T.reshape(_K, _NH, _FH).transpose(1, 0, 2).reshape(-1)
    run = pl.kernel(
        _sc_body,
        mesh=plsc.VectorSubcoreMesh(core_axis_name="c", subcore_axis_name="s"),
        out_type=jax.ShapeDtypeStruct((_B * _N,), jnp.float32),
        scratch_types=(
            [
                pltpu.VMEM((_RPW * _K,), jnp.float32),
                pltpu.VMEM((_K * _FH,), jnp.float32),
                pltpu.VMEM((_FH,), jnp.float32),
                pltpu.VMEM((_NSLOT * _FH,), jnp.float32)
            ] + [pltpu.SemaphoreType.DMA] * _NSLOT
        ),
    )
    out = run(condition.reshape(-1), wt, b)
    return out.reshape(_B, 32, 16, 16)
